# baseline (speedup 1.0000x reference)
import functools

import jax
import jax.numpy as jnp
from jax import lax
from jax.experimental import pallas as pl
from jax.experimental.pallas import tpu as pltpu
from jax.experimental.pallas import tpu_sc as plsc

N_NODES = 10000
N_EDGES = 320000
D = 128
DH = D // 2

NC = 2
NS = 16

K = 128
NBUF = 5
LOOK = 3
NIB = 10
NCHUNK = 160
ET = NCHUNK * K
E_PAD = NS * ET
N_PAD = 10112
ROWS_PER_TILE = N_PAD // NS

_mesh = plsc.VectorSubcoreMesh(core_axis_name="c", subcore_axis_name="s")


@functools.partial(
    pl.kernel,
    out_type=jax.ShapeDtypeStruct((NC, N_PAD, DH), jnp.float32),
    mesh=_mesh,
    compiler_params=pltpu.CompilerParams(
        use_tc_tiling_on_sc=False, needs_layout_passes=False),
    scratch_types=[
        pltpu.VMEM((NIB, 3, K), jnp.int32),
    ]
      + [pltpu.VMEM((K, DH), jnp.float32)] * NBUF
      + [pltpu.VMEM_SHARED((N_PAD, DH), jnp.float32),
         pltpu.VMEM_SHARED((N_PAD, DH), jnp.float32)]
      + [pltpu.SemaphoreType.DMA] * (NIB + 2 * NBUF),
)
def _sc_spmm(xs_hbm, edges_hbm, out_hbm, idx_v, *rest):
    bufs = rest[:NBUF]
    xtab_sh = rest[NBUF]
    acc_sh = rest[NBUF + 1]
    isems = rest[NBUF + 2:NBUF + 2 + NIB]
    gsems = rest[NBUF + 2 + NIB:NBUF + 2 + NIB + NBUF]
    ssems = rest[NBUF + 2 + NIB + NBUF:]
    c = lax.axis_index("c")
    s = lax.axis_index("s")

    base = s * ROWS_PER_TILE
    pltpu.sync_copy(xs_hbm.at[pl.ds(c * N_PAD + base, ROWS_PER_TILE)],
                    xtab_sh.at[pl.ds(base, ROWS_PER_TILE)])

    zero16 = jnp.zeros((16,), jnp.float32)

    def _zero_body(e, carry):
        for v in range(DH // 16):
            bufs[0][e, pl.ds(v * 16, 16)] = zero16
        return carry

    lax.fori_loop(0, K, _zero_body, 0, unroll=False)
    nfull, tail = divmod(ROWS_PER_TILE, K)
    for i in range(nfull):
        pltpu.sync_copy(bufs[0], acc_sh.at[pl.ds(base + i * K, K)])
    if tail:
        pltpu.sync_copy(bufs[0].at[pl.ds(0, tail)],
                        acc_sh.at[pl.ds(base + nfull * K, tail)])
    plsc.subcore_barrier()

    def _issue_idx(j, islot):
        pltpu.async_copy(edges_hbm.at[s, j], idx_v.at[islot], isems[islot])

    def _wait_idx(j, islot):
        pltpu.make_async_copy(
            edges_hbm.at[s, j], idx_v.at[islot], isems[islot]).wait()

    def _issue_gather(islot, slot):
        pltpu.async_copy(xtab_sh.at[idx_v.at[islot, 1]], bufs[slot],
                         gsems[slot])

    def _wait_gather(islot, slot):
        pltpu.make_async_copy(xtab_sh.at[idx_v.at[islot, 1]], bufs[slot],
                              gsems[slot]).wait()

    def _issue_scatter(islot, slot):
        pltpu.async_copy(bufs[slot], acc_sh.at[idx_v.at[islot, 0]],
                         ssems[slot], add=True)

    def _wait_scatter(islot, slot):
        pltpu.make_async_copy(bufs[slot], acc_sh.at[idx_v.at[islot, 0]],
                              ssems[slot]).wait()

    def _scale(islot, slot):
        def _scale_body(g, inner):
            a16 = plsc.bitcast(
                idx_v[islot, 2, pl.ds(g * 16, 16)], jnp.float32)
            for l in range(16):
                a = a16[l]
                e = g * 16 + l
                for v in range(DH // 16):
                    sl = pl.ds(v * 16, 16)
                    bufs[slot][e, sl] = bufs[slot][e, sl] * a
            return inner

        lax.fori_loop(0, K // 16, _scale_body, 0, unroll=False)

    for jj in range(NIB):
        _issue_idx(jj, jj)
    for jj in range(LOOK):
        _wait_idx(jj, jj)
        _issue_gather(jj, jj)

    def _outer(i, carry):
        for b in range(NIB):
            j = NIB * i + b
            db = b % NBUF
            bt = (b + LOOK) % NBUF
            it = (b + LOOK) % NIB
            ir = (b + LOOK - NBUF) % NIB

            @pl.when(j + LOOK < NCHUNK)
            def _():
                @pl.when(j + LOOK >= NBUF)
                def _():
                    _wait_scatter(ir, bt)

                    @pl.when(j + LOOK - NBUF + NIB < NCHUNK)
                    def _():
                        _issue_idx(j + LOOK - NBUF + NIB, ir)

                _wait_idx(j + LOOK, it)
                _issue_gather(it, bt)

            _wait_gather(b % NIB, db)
            _scale(b % NIB, db)
            _issue_scatter(b % NIB, db)
        return carry

    lax.fori_loop(0, NCHUNK // NIB, _outer, 0, unroll=False)

    for jj in range(NCHUNK - NBUF, NCHUNK):
        _wait_scatter(jj % NIB, jj % NBUF)
    plsc.subcore_barrier()

    pltpu.sync_copy(acc_sh.at[pl.ds(base, ROWS_PER_TILE)],
                    out_hbm.at[c, pl.ds(base, ROWS_PER_TILE)])


def _combine_body(p_ref, o_ref):
    o_ref[...] = jnp.maximum(
        jnp.concatenate([p_ref[0], p_ref[1]], axis=-1), 0.0)


_combine = pl.pallas_call(
    _combine_body,
    out_shape=jax.ShapeDtypeStruct((N_NODES, D), jnp.float32),
    grid=(10,),
    in_specs=[pl.BlockSpec((2, N_NODES // 10, DH), lambda i: (0, i, 0))],
    out_specs=pl.BlockSpec((N_NODES // 10, D), lambda i: (i, 0)),
)


def kernel(t, x, edge_index, A_vals):
    zrow = jnp.zeros((N_PAD - N_NODES, DH), jnp.float32)
    xs = jnp.concatenate([x[:, :DH], zrow, x[:, DH:], zrow], axis=0)
    pad = E_PAD - N_EDGES
    zpad_i = jnp.zeros((pad,), jnp.int32)
    row = jnp.concatenate([edge_index[0], zpad_i]).reshape(NS, NCHUNK, K)
    col = jnp.concatenate([edge_index[1], zpad_i]).reshape(NS, NCHUNK, K)
    bits = jnp.concatenate(
        [lax.bitcast_convert_type(A_vals, jnp.int32), zpad_i]
    ).reshape(NS, NCHUNK, K)
    edges = jnp.stack([row, col, bits], axis=2)
    partials = _sc_spmm(xs, edges)
    return _combine(partials)


# speedup vs baseline: 1.1788x; 1.1788x over previous
import functools

import jax
import jax.numpy as jnp
from jax import lax
from jax.experimental import pallas as pl
from jax.experimental.pallas import tpu as pltpu
from jax.experimental.pallas import tpu_sc as plsc

N_NODES = 10000
N_EDGES = 320000
D = 128
DH = D // 2

NC = 2
NS = 16

K = 128
NBUF = 3
LOOK = 1
NCHUNK = 159
ET = NCHUNK * K
E_PAD = NS * ET
N_PAD = 10240
ROWS_PER_TILE = N_PAD // NS

_mesh = plsc.VectorSubcoreMesh(core_axis_name="c", subcore_axis_name="s")


@functools.partial(
    pl.kernel,
    out_type=jax.ShapeDtypeStruct((N_NODES, NC, DH), jnp.float32),
    mesh=_mesh,
    compiler_params=pltpu.CompilerParams(use_tc_tiling_on_sc=False),
    scratch_types=[
        pltpu.VMEM((NCHUNK, K), jnp.int32),
        pltpu.VMEM((NCHUNK, K), jnp.int32),
        pltpu.VMEM((NCHUNK, K), jnp.float32),
    ]
      + [pltpu.VMEM((K, DH), jnp.float32)] * NBUF
      + [pltpu.VMEM_SHARED((N_PAD, DH), jnp.float32)]
      + [pltpu.SemaphoreType.DMA] * (2 * NBUF),
)
def _sc_spmm(xs_hbm, col_hbm, row_hbm, vals_hbm, out_hbm,
             col_v, row_v, vals_v, *rest):
    bufs = rest[:NBUF]
    acc_sh = rest[NBUF]
    gsems = rest[NBUF + 1:NBUF + 1 + NBUF]
    ssems = rest[NBUF + 1 + NBUF:]
    c = lax.axis_index("c")
    s = lax.axis_index("s")

    pltpu.sync_copy(col_hbm.at[s], col_v)
    pltpu.sync_copy(row_hbm.at[s], row_v)
    pltpu.sync_copy(vals_hbm.at[s], vals_v)

    coff = c * N_NODES

    def _off_body(j, carry):
        for g in range(K // 16):
            sl = pl.ds(g * 16, 16)
            col_v[j, sl] = col_v[j, sl] + coff
        return carry

    lax.fori_loop(0, NCHUNK, _off_body, 0, unroll=False)

    zero16 = jnp.zeros((16,), jnp.float32)

    def _zero_body(e, carry):
        for v in range(DH // 16):
            bufs[0][e, pl.ds(v * 16, 16)] = zero16
        return carry

    lax.fori_loop(0, K, _zero_body, 0, unroll=False)
    base = s * ROWS_PER_TILE
    for i in range(ROWS_PER_TILE // K):
        pltpu.sync_copy(bufs[0], acc_sh.at[pl.ds(base + i * K, K)])
    plsc.subcore_barrier()

    def _issue_gather(j, slot):
        pltpu.async_copy(xs_hbm.at[col_v.at[j]], bufs[slot], gsems[slot])

    def _wait_gather(j, slot):
        pltpu.make_async_copy(
            xs_hbm.at[col_v.at[j]], bufs[slot], gsems[slot]).wait()

    def _issue_scatter(j, slot):
        pltpu.async_copy(bufs[slot], acc_sh.at[row_v.at[j]],
                         ssems[slot], add=True)

    def _wait_scatter(j, slot):
        pltpu.make_async_copy(bufs[slot], acc_sh.at[row_v.at[j]],
                              ssems[slot]).wait()

    def _scale(j, slot):
        def _scale_body(g, inner):
            a16 = vals_v[j, pl.ds(g * 16, 16)]
            for l in range(16):
                a = a16[l]
                e = g * 16 + l
                for v in range(DH // 16):
                    sl = pl.ds(v * 16, 16)
                    bufs[slot][e, sl] = bufs[slot][e, sl] * a
            return inner

        lax.fori_loop(0, K // 16, _scale_body, 0, unroll=False)

    for jj in range(LOOK):
        _issue_gather(jj, jj)

    def _outer(i, carry):
        for b in range(NBUF):
            j = NBUF * i + b
            bt = (b + LOOK) % NBUF

            @pl.when(j + LOOK < NCHUNK)
            def _():
                @pl.when(j + LOOK >= NBUF)
                def _():
                    _wait_scatter(j + LOOK - NBUF, bt)

                _issue_gather(j + LOOK, bt)

            _wait_gather(j, b)
            _scale(j, b)
            _issue_scatter(j, b)
        return carry

    lax.fori_loop(0, NCHUNK // NBUF, _outer, 0, unroll=False)

    for jj in range(NCHUNK - NBUF, NCHUNK):
        _wait_scatter(jj, jj % NBUF)
    plsc.subcore_barrier()


    def _relu_pass(nrows, dst_base):
        pltpu.sync_copy(acc_sh.at[pl.ds(dst_base, nrows)],
                        bufs[0].at[pl.ds(0, nrows)])

        def _relu_body(e, carry):
            for v in range(DH // 16):
                sl = pl.ds(v * 16, 16)
                bufs[0][e, sl] = jnp.maximum(bufs[0][e, sl], 0.0)
            return carry

        lax.fori_loop(0, nrows, _relu_body, 0, unroll=False)
        pltpu.sync_copy(bufs[0].at[pl.ds(0, nrows)],
                        out_hbm.at[pl.ds(dst_base, nrows), c])

    for i in range(ROWS_PER_TILE // K):
        blk = base + i * K

        @pl.when(blk <= N_NODES - K)
        def _():
            _relu_pass(K, blk)

    @pl.when(s == NS - 1)
    def _():
        _relu_pass(K, N_NODES - K)


def kernel(t, x, edge_index, A_vals):
    xs = jnp.concatenate([x[:, :DH], x[:, DH:]], axis=0)
    pad = E_PAD - N_EDGES
    zpad_i = jnp.zeros((pad,), jnp.int32)
    row = jnp.concatenate([edge_index[0], zpad_i]).reshape(NS, NCHUNK, K)
    col = jnp.concatenate([edge_index[1], zpad_i]).reshape(NS, NCHUNK, K)
    vals = jnp.concatenate(
        [A_vals, jnp.zeros((pad,), jnp.float32)]).reshape(NS, NCHUNK, K)
    out = _sc_spmm(xs, col, row, vals)
    return out.reshape(N_NODES, D)


# speedup vs baseline: 1.2394x; 1.0514x over previous
import functools

import jax
import jax.numpy as jnp
from jax import lax
from jax.experimental import pallas as pl
from jax.experimental.pallas import tpu as pltpu
from jax.experimental.pallas import tpu_sc as plsc

N_NODES = 10000
N_EDGES = 320000
D = 128
DH = D // 2

NC = 2
NS = 16

K = 128
NBUF = 3
LOOK = 1
NCHUNK = 159
ET = NCHUNK * K
E_PAD = NS * ET
N_PAD = 10240
ROWS_PER_TILE = N_PAD // NS

_mesh = plsc.VectorSubcoreMesh(core_axis_name="c", subcore_axis_name="s")


@functools.partial(
    pl.kernel,
    out_type=jax.ShapeDtypeStruct((NC, N_PAD, DH), jnp.float32),
    mesh=_mesh,
    compiler_params=pltpu.CompilerParams(use_tc_tiling_on_sc=False),
    scratch_types=[
        pltpu.VMEM((NCHUNK, K), jnp.int32),
        pltpu.VMEM((NCHUNK, K), jnp.int32),
        pltpu.VMEM((NCHUNK, K), jnp.float32),
    ]
      + [pltpu.VMEM((K, DH), jnp.float32)] * NBUF
      + [pltpu.VMEM_SHARED((N_PAD, DH), jnp.float32)]
      + [pltpu.SemaphoreType.DMA] * (2 * NBUF),
)
def _sc_spmm(xs_hbm, col_hbm, row_hbm, vals_hbm, out_hbm,
             col_v, row_v, vals_v, *rest):
    bufs = rest[:NBUF]
    acc_sh = rest[NBUF]
    gsems = rest[NBUF + 1:NBUF + 1 + NBUF]
    ssems = rest[NBUF + 1 + NBUF:]
    c = lax.axis_index("c")
    s = lax.axis_index("s")

    pltpu.sync_copy(col_hbm.at[s], col_v)
    pltpu.sync_copy(row_hbm.at[s], row_v)
    pltpu.sync_copy(vals_hbm.at[s], vals_v)

    coff = c * N_NODES

    def _off_body(j, carry):
        for g in range(K // 16):
            sl = pl.ds(g * 16, 16)
            col_v[j, sl] = col_v[j, sl] + coff
        return carry

    lax.fori_loop(0, NCHUNK, _off_body, 0, unroll=False)

    zero16 = jnp.zeros((16,), jnp.float32)

    def _zero_body(e, carry):
        for v in range(DH // 16):
            bufs[0][e, pl.ds(v * 16, 16)] = zero16
        return carry

    lax.fori_loop(0, K, _zero_body, 0, unroll=False)
    base = s * ROWS_PER_TILE
    for i in range(ROWS_PER_TILE // K):
        pltpu.sync_copy(bufs[0], acc_sh.at[pl.ds(base + i * K, K)])
    plsc.subcore_barrier()

    def _issue_gather(j, slot):
        pltpu.async_copy(xs_hbm.at[col_v.at[j]], bufs[slot], gsems[slot])

    def _wait_gather(j, slot):
        pltpu.make_async_copy(
            xs_hbm.at[col_v.at[j]], bufs[slot], gsems[slot]).wait()

    def _issue_scatter(j, slot):
        pltpu.async_copy(bufs[slot], acc_sh.at[row_v.at[j]],
                         ssems[slot], add=True)

    def _wait_scatter(j, slot):
        pltpu.make_async_copy(bufs[slot], acc_sh.at[row_v.at[j]],
                              ssems[slot]).wait()

    def _scale(j, slot):
        def _scale_body(g, inner):
            a16 = vals_v[j, pl.ds(g * 16, 16)]
            for l in range(16):
                a = a16[l]
                e = g * 16 + l
                for v in range(DH // 16):
                    sl = pl.ds(v * 16, 16)
                    bufs[slot][e, sl] = bufs[slot][e, sl] * a
            return inner

        lax.fori_loop(0, K // 16, _scale_body, 0, unroll=False)

    for jj in range(LOOK):
        _issue_gather(jj, jj)

    def _outer(i, carry):
        for b in range(NBUF):
            j = NBUF * i + b
            bt = (b + LOOK) % NBUF

            @pl.when(j + LOOK < NCHUNK)
            def _():
                @pl.when(j + LOOK >= NBUF)
                def _():
                    _wait_scatter(j + LOOK - NBUF, bt)

                _issue_gather(j + LOOK, bt)

            _wait_gather(j, b)
            _scale(j, b)
            _issue_scatter(j, b)
        return carry

    lax.fori_loop(0, NCHUNK // NBUF, _outer, 0, unroll=False)

    for jj in range(NCHUNK - NBUF, NCHUNK):
        _wait_scatter(jj, jj % NBUF)
    plsc.subcore_barrier()

    pltpu.sync_copy(acc_sh.at[pl.ds(base, ROWS_PER_TILE)],
                    out_hbm.at[c, pl.ds(base, ROWS_PER_TILE)])


def _combine_body(p_ref, o_ref):
    o_ref[...] = jnp.maximum(
        jnp.concatenate([p_ref[0], p_ref[1]], axis=-1), 0.0)


_combine = pl.pallas_call(
    _combine_body,
    out_shape=jax.ShapeDtypeStruct((N_NODES, D), jnp.float32),
    grid=(10,),
    in_specs=[pl.BlockSpec((2, N_NODES // 10, DH), lambda i: (0, i, 0))],
    out_specs=pl.BlockSpec((N_NODES // 10, D), lambda i: (i, 0)),
)


def kernel(t, x, edge_index, A_vals):
    xs = jnp.concatenate([x[:, :DH], x[:, DH:]], axis=0)
    pad = E_PAD - N_EDGES
    zpad_i = jnp.zeros((pad,), jnp.int32)
    row = jnp.concatenate([edge_index[0], zpad_i]).reshape(NS, NCHUNK, K)
    col = jnp.concatenate([edge_index[1], zpad_i]).reshape(NS, NCHUNK, K)
    vals = jnp.concatenate(
        [A_vals, jnp.zeros((pad,), jnp.float32)]).reshape(NS, NCHUNK, K)
    partials = _sc_spmm(xs, col, row, vals)
    return _combine(partials)


# speedup vs baseline: 1.2573x; 1.0145x over previous
import functools

import jax
import jax.numpy as jnp
from jax import lax
from jax.experimental import pallas as pl
from jax.experimental.pallas import tpu as pltpu
from jax.experimental.pallas import tpu_sc as plsc

N_NODES = 10000
N_EDGES = 320000
D = 128
DH = D // 2

NC = 2
NS = 16

K = 128
NBUF = 3
LOOK = 1
NCHUNK = 159
ET = NCHUNK * K
E_PAD = NS * ET
N_PAD = 10240
ROWS_PER_TILE = N_PAD // NS

_mesh = plsc.VectorSubcoreMesh(core_axis_name="c", subcore_axis_name="s")


@functools.partial(
    pl.kernel,
    out_type=jax.ShapeDtypeStruct((NC, N_PAD, DH), jnp.float32),
    mesh=_mesh,
    compiler_params=pltpu.CompilerParams(use_tc_tiling_on_sc=False),
    scratch_types=[
        pltpu.VMEM((NCHUNK, K), jnp.int32),
        pltpu.VMEM((NCHUNK, K), jnp.int32),
        pltpu.VMEM((NCHUNK, K), jnp.float32),
    ]
      + [pltpu.VMEM((K, DH), jnp.float32)] * NBUF
      + [pltpu.VMEM_SHARED((N_PAD, DH), jnp.float32)]
      + [pltpu.SemaphoreType.DMA] * (2 * NBUF),
)
def _sc_spmm(xs_hbm, col_hbm, row_hbm, vals_hbm, out_hbm,
             col_v, row_v, vals_v, *rest):
    bufs = rest[:NBUF]
    acc_sh = rest[NBUF]
    gsems = rest[NBUF + 1:NBUF + 1 + NBUF]
    ssems = rest[NBUF + 1 + NBUF:]
    c = lax.axis_index("c")
    s = lax.axis_index("s")

    cp1 = pltpu.async_copy(col_hbm.at[s], col_v, gsems[0])
    cp2 = pltpu.async_copy(row_hbm.at[s], row_v, gsems[1])
    cp3 = pltpu.async_copy(vals_hbm.at[s], vals_v, gsems[2])
    cp1.wait()
    cp2.wait()
    cp3.wait()

    coff = c * N_NODES

    def _off_body(j, carry):
        for g in range(K // 16):
            sl = pl.ds(g * 16, 16)
            col_v[j, sl] = col_v[j, sl] + coff
        return carry

    lax.fori_loop(0, NCHUNK, _off_body, 0, unroll=False)

    zero16 = jnp.zeros((16,), jnp.float32)

    def _zero_body(e, carry):
        for v in range(DH // 16):
            bufs[0][e, pl.ds(v * 16, 16)] = zero16
        return carry

    lax.fori_loop(0, K, _zero_body, 0, unroll=False)
    base = s * ROWS_PER_TILE
    for i in range(ROWS_PER_TILE // K):
        pltpu.sync_copy(bufs[0], acc_sh.at[pl.ds(base + i * K, K)])
    plsc.subcore_barrier()

    def _issue_gather(j, slot):
        pltpu.async_copy(xs_hbm.at[col_v.at[j]], bufs[slot], gsems[slot])

    def _wait_gather(j, slot):
        pltpu.make_async_copy(
            xs_hbm.at[col_v.at[j]], bufs[slot], gsems[slot]).wait()

    def _issue_scatter(j, slot):
        pltpu.async_copy(bufs[slot], acc_sh.at[row_v.at[j]],
                         ssems[slot], add=True)

    def _wait_scatter(j, slot):
        pltpu.make_async_copy(bufs[slot], acc_sh.at[row_v.at[j]],
                              ssems[slot]).wait()

    def _scale(j, slot):
        def _scale_body(g, inner):
            a16 = vals_v[j, pl.ds(g * 16, 16)]
            for l in range(16):
                a = a16[l]
                e = g * 16 + l
                for v in range(DH // 16):
                    sl = pl.ds(v * 16, 16)
                    bufs[slot][e, sl] = bufs[slot][e, sl] * a
            return inner

        lax.fori_loop(0, K // 16, _scale_body, 0, unroll=True)

    for jj in range(LOOK):
        _issue_gather(jj, jj)

    def _outer(i, carry):
        for b in range(NBUF):
            j = NBUF * i + b
            bt = (b + LOOK) % NBUF

            @pl.when(j + LOOK < NCHUNK)
            def _():
                @pl.when(j + LOOK >= NBUF)
                def _():
                    _wait_scatter(j + LOOK - NBUF, bt)

                _issue_gather(j + LOOK, bt)

            _wait_gather(j, b)
            _scale(j, b)
            _issue_scatter(j, b)
        return carry

    lax.fori_loop(0, NCHUNK // NBUF, _outer, 0, unroll=False)

    for jj in range(NCHUNK - NBUF, NCHUNK):
        _wait_scatter(jj, jj % NBUF)
    plsc.subcore_barrier()

    pltpu.sync_copy(acc_sh.at[pl.ds(base, ROWS_PER_TILE)],
                    out_hbm.at[c, pl.ds(base, ROWS_PER_TILE)])


def _combine_body(p_ref, o_ref):
    o_ref[...] = jnp.maximum(
        jnp.concatenate([p_ref[0], p_ref[1]], axis=-1), 0.0)


_combine = pl.pallas_call(
    _combine_body,
    out_shape=jax.ShapeDtypeStruct((N_NODES, D), jnp.float32),
    grid=(10,),
    in_specs=[pl.BlockSpec((2, N_NODES // 10, DH), lambda i: (0, i, 0))],
    out_specs=pl.BlockSpec((N_NODES // 10, D), lambda i: (i, 0)),
)


def kernel(t, x, edge_index, A_vals):
    xs = jnp.concatenate([x[:, :DH], x[:, DH:]], axis=0)
    pad = E_PAD - N_EDGES
    zpad_i = jnp.zeros((pad,), jnp.int32)
    row = jnp.concatenate([edge_index[0], zpad_i]).reshape(NS, NCHUNK, K)
    col = jnp.concatenate([edge_index[1], zpad_i]).reshape(NS, NCHUNK, K)
    vals = jnp.concatenate(
        [A_vals, jnp.zeros((pad,), jnp.float32)]).reshape(NS, NCHUNK, K)
    partials = _sc_spmm(xs, col, row, vals)
    return _combine(partials)
